# Initial kernel scaffold; baseline (speedup 1.0000x reference)
import jax
import jax.numpy as jnp
from jax.experimental import pallas as pl


def kernel(x, edge_index, edge_attr, batch, We, be, lin1_W, lin1_b, lin2_W, lin2_b, gru_Wih, gru_Whh, gru_bih, gru_bhh, mol_Wl, mol_Wr, mol_att, mol_bias, molgru_Wih, molgru_Whh, molgru_bih, molgru_bhh, lin_W, lin_b):
    raise NotImplementedError("write your pallas kernel here")


# speedup vs baseline: 1.0654x; 1.0654x over previous
import math

import jax
import jax.numpy as jnp
from jax import lax
from jax.experimental import pallas as pl
from jax.experimental.pallas import tpu as pltpu

N = 10000
E = 320000
D = 128
ED = 16
L = 3
G = 128
STEPS = 2

_BN = float(jnp.float32(1.0) / jnp.float32(math.sqrt(1.0 + 1e-5)))


def _leaky(v, s=0.01):
    return jnp.where(v >= 0, v, s * v)


_EE_BLK = 8000


def _ee_body(ea_ref, w_ref, b_ref, out_ref):
    out_ref[0] = (
        jnp.dot(ea_ref[...], w_ref[0], preferred_element_type=jnp.float32)
        + b_ref[0]
    )


def _edge_emb(edge_attr, We, be3):
    return pl.pallas_call(
        _ee_body,
        grid=(L, E // _EE_BLK),
        in_specs=[
            pl.BlockSpec((_EE_BLK, ED), lambda l, i: (i, 0)),
            pl.BlockSpec((1, ED, D), lambda l, i: (l, 0, 0)),
            pl.BlockSpec((1, 1, D), lambda l, i: (l, 0, 0)),
        ],
        out_specs=pl.BlockSpec((1, _EE_BLK, D), lambda l, i: (l, i, 0)),
        out_shape=jax.ShapeDtypeStruct((L, E, D), jnp.float32),
    )(edge_attr, We, be3)


_DL_BLK = 1000


def _dense_body(x_ref, a_ref, w1_ref, b1_ref, w2_ref, b2_ref,
                wih_ref, whh_ref, bih_ref, bhh_ref, o_ref):
    x = x_ref[...]
    t = x + a_ref[...]
    t = jnp.dot(t, w1_ref[...], preferred_element_type=jnp.float32) + b1_ref[...]
    t = _leaky(t * _BN)
    h = jnp.dot(t, w2_ref[...], preferred_element_type=jnp.float32) + b2_ref[...]
    h = jnp.where(h > 0, h, jnp.exp(h) - 1.0)
    gi = jnp.dot(h, wih_ref[...], preferred_element_type=jnp.float32) + bih_ref[...]
    gh = jnp.dot(x, whh_ref[...], preferred_element_type=jnp.float32) + bhh_ref[...]
    r = jax.nn.sigmoid(gi[:, :D] + gh[:, :D])
    z = jax.nn.sigmoid(gi[:, D:2 * D] + gh[:, D:2 * D])
    n = jnp.tanh(gi[:, 2 * D:] + r * gh[:, 2 * D:])
    o_ref[...] = _leaky((1.0 - z) * n + z * x)


def _dense_layer(x, aggr, w1, b1, w2, b2, wih, whh, bih, bhh):
    full = lambda s: pl.BlockSpec(s, lambda i: tuple(0 for _ in s))
    return pl.pallas_call(
        _dense_body,
        grid=(N // _DL_BLK,),
        in_specs=[
            pl.BlockSpec((_DL_BLK, D), lambda i: (i, 0)),
            pl.BlockSpec((_DL_BLK, D), lambda i: (i, 0)),
            full((D, D)), full((1, D)), full((D, D)), full((1, D)),
            full((D, 3 * D)), full((D, 3 * D)), full((1, 3 * D)), full((1, 3 * D)),
        ],
        out_specs=pl.BlockSpec((_DL_BLK, D), lambda i: (i, 0)),
        out_shape=jax.ShapeDtypeStruct((N, D), jnp.float32),
    )(x, aggr, w1, b1.reshape(1, D), w2, b2.reshape(1, D),
      wih, whh, bih.reshape(1, 3 * D), bhh.reshape(1, 3 * D))


def _readout_body(x_ref, b_ref, wl_ref, wr_ref, att_ref, bias_ref,
                  wih_ref, whh_ref, bih_ref, bhh_ref, lw_ref, lb_ref, o_ref):
    x = x_ref[...]
    oh = (b_ref[...] == lax.broadcasted_iota(jnp.int32, (N, G), 1)).astype(
        jnp.float32)
    dn = (((0,), (0,)), ((), ()))
    pool = lax.dot_general(oh, x, dn, preferred_element_type=jnp.float32)
    out = _leaky(pool)
    xl = jnp.dot(x, wl_ref[...], preferred_element_type=jnp.float32)
    att = att_ref[...]
    for _ in range(STEPS):
        xr = jnp.dot(out, wr_ref[...], preferred_element_type=jnp.float32)
        z = xl + jnp.dot(oh, xr, preferred_element_type=jnp.float32)
        z = jnp.where(z >= 0, z, 0.2 * z)
        e = jnp.sum(z * att, axis=1, keepdims=True)
        m = jnp.max(jnp.where(oh > 0, e, -jnp.inf), axis=0, keepdims=True)
        m = jnp.where(jnp.isfinite(m), m, 0.0)
        ex = jnp.exp(e - jnp.sum(oh * m, axis=1, keepdims=True))
        den = lax.dot_general(oh, ex, dn, preferred_element_type=jnp.float32)
        den_b = jnp.dot(oh, den, preferred_element_type=jnp.float32)
        alpha = ex / jnp.maximum(den_b, 1e-16)
        h = lax.dot_general(oh, alpha * xl, dn,
                            preferred_element_type=jnp.float32) + bias_ref[...]
        h = jnp.where(h > 0, h, jnp.exp(h) - 1.0)
        gi = jnp.dot(h, wih_ref[...], preferred_element_type=jnp.float32) \
            + bih_ref[...]
        gh = jnp.dot(out, whh_ref[...], preferred_element_type=jnp.float32) \
            + bhh_ref[...]
        r = jax.nn.sigmoid(gi[:, :D] + gh[:, :D])
        zz = jax.nn.sigmoid(gi[:, D:2 * D] + gh[:, D:2 * D])
        n = jnp.tanh(gi[:, 2 * D:] + r * gh[:, 2 * D:])
        out = _leaky((1.0 - zz) * n + zz * out)
    o_ref[...] = jnp.dot(out, lw_ref[...], preferred_element_type=jnp.float32) \
        + lb_ref[...]


def _readout(x, batch, mol_Wl, mol_Wr, mol_att, mol_bias,
             molgru_Wih, molgru_Whh, molgru_bih, molgru_bhh, lin_W, lin_b):
    full = lambda s: pl.BlockSpec(s, lambda: tuple(0 for _ in s))
    return pl.pallas_call(
        _readout_body,
        in_specs=[
            full((N, D)), full((N, 1)),
            full((D, D)), full((D, D)), full((1, D)), full((1, D)),
            full((D, 3 * D)), full((D, 3 * D)), full((1, 3 * D)),
            full((1, 3 * D)), full((D, D)), full((1, D)),
        ],
        out_specs=full((G, D)),
        out_shape=jax.ShapeDtypeStruct((G, D), jnp.float32),
    )(x, batch.reshape(N, 1), mol_Wl, mol_Wr, mol_att.reshape(1, D),
      mol_bias.reshape(1, D), molgru_Wih, molgru_Whh,
      molgru_bih.reshape(1, 3 * D), molgru_bhh.reshape(1, 3 * D),
      lin_W, lin_b.reshape(1, D))


def kernel(x, edge_index, edge_attr, batch, We, be, lin1_W, lin1_b, lin2_W,
           lin2_b, gru_Wih, gru_Whh, gru_bih, gru_bhh, mol_Wl, mol_Wr,
           mol_att, mol_bias, molgru_Wih, molgru_Whh, molgru_bih, molgru_bhh,
           lin_W, lin_b):
    src = edge_index[0]
    dst = edge_index[1]
    e_emb3 = _edge_emb(edge_attr, We, be.reshape(L, 1, D))
    for l in range(L):
        msg = jax.nn.relu(x[src] + e_emb3[l])
        aggr = jax.ops.segment_sum(msg, dst, num_segments=N)
        x = _dense_layer(x, aggr, lin1_W[l], lin1_b[l], lin2_W[l], lin2_b[l],
                         gru_Wih[l], gru_Whh[l], gru_bih[l], gru_bhh[l])
    return _readout(x, batch, mol_Wl, mol_Wr, mol_att, mol_bias,
                    molgru_Wih, molgru_Whh, molgru_bih, molgru_bhh,
                    lin_W, lin_b)


# speedup vs baseline: 2.1515x; 2.0193x over previous
import functools
import math

import jax
import jax.numpy as jnp
from jax import lax
from jax.experimental import pallas as pl
from jax.experimental.pallas import tpu as pltpu
from jax.experimental.pallas import tpu_sc as plsc

N = 10000
E = 320000
D = 128
ED = 16
L = 3
G = 128
STEPS = 2

_NCORE = 2
_NSUB = 16
_NW = _NCORE * _NSUB
_CHUNK = 128
_NCHUNK = 80
_EPW = _NCHUNK * _CHUNK
_EP = _EPW * _NW
_RPS = 632
_NACC = _RPS * _NSUB

_BN = 1.0 / math.sqrt(1.0 + 1e-5)


def _leaky(v, s=0.01):
    return jnp.where(v >= 0, v, s * v)


_EE_BLK = 8192


def _ee_body(ea_ref, w_ref, b_ref, out_ref):
    out_ref[0] = (
        jnp.dot(ea_ref[...], w_ref[0], preferred_element_type=jnp.float32)
        + b_ref[0]
    )


def _edge_emb(edge_attr_pad, We, be3):
    return pl.pallas_call(
        _ee_body,
        grid=(L, _EP // _EE_BLK),
        in_specs=[
            pl.BlockSpec((_EE_BLK, ED), lambda l, i: (i, 0)),
            pl.BlockSpec((1, ED, D), lambda l, i: (l, 0, 0)),
            pl.BlockSpec((1, 1, D), lambda l, i: (l, 0, 0)),
        ],
        out_specs=pl.BlockSpec((1, _EE_BLK, D), lambda l, i: (l, i, 0)),
        out_shape=jax.ShapeDtypeStruct((L, _EP, D), jnp.float32),
    )(edge_attr_pad, We, be3)


def _sc_body(l, x_hbm, ee_hbm, src_hbm, dst_hbm, out_hbm,
             srcv, dstv, buf, acc, sem_lin, sem_gat, sem_sc):
    c = lax.axis_index("c")
    s = lax.axis_index("s")
    woff = (c * _NSUB + s) * _EPW
    r0 = s * _RPS

    zero16 = jnp.zeros((16,), jnp.float32)

    def zrow(r, carry):
        for k in range(8):
            buf[0, r, pl.ds(k * 16, 16)] = zero16
        return carry

    lax.fori_loop(0, _CHUNK, zrow, 0)
    for j in range(4):
        pltpu.sync_copy(buf.at[0], acc.at[pl.ds(r0 + j * 128, 128)])
    pltpu.sync_copy(buf.at[0, pl.ds(0, _RPS - 512)],
                    acc.at[pl.ds(r0 + 512, _RPS - 512)])
    plsc.subcore_barrier()

    def step(i, carry):
        base = woff + i * _CHUNK
        d1 = pltpu.async_copy(src_hbm.at[pl.ds(base, _CHUNK)], srcv.at[0],
                              sem_lin.at[0])
        d2 = pltpu.async_copy(dst_hbm.at[pl.ds(base, _CHUNK)], dstv.at[0],
                              sem_lin.at[0])
        d3 = pltpu.async_copy(ee_hbm.at[l, pl.ds(base, _CHUNK)], buf.at[0],
                              sem_lin.at[0])
        d1.wait()
        d2.wait()
        d3.wait()
        pltpu.async_copy(x_hbm.at[srcv.at[0]], buf.at[0], sem_gat.at[0],
                         add=True).wait()

        def relu_row(r, rcarry):
            for k in range(8):
                v = buf[0, r, pl.ds(k * 16, 16)]
                buf[0, r, pl.ds(k * 16, 16)] = jnp.maximum(v, 0.0)
            return rcarry

        lax.fori_loop(0, _CHUNK, relu_row, 0)
        pltpu.async_copy(buf.at[0], acc.at[dstv.at[0]], sem_sc.at[0],
                         add=True).wait()
        return carry

    lax.fori_loop(0, _NCHUNK, step, 0)
    plsc.subcore_barrier()

    for j in range(4):
        pltpu.sync_copy(acc.at[pl.ds(r0 + j * 128, 128)],
                        out_hbm.at[c, pl.ds(r0 + j * 128, 128)])
    pltpu.sync_copy(acc.at[pl.ds(r0 + 512, _RPS - 512)],
                    out_hbm.at[c, pl.ds(r0 + 512, _RPS - 512)])


def _sc_edge(l, x, ee, src_pad, dst_pad):
    return pl.kernel(
        functools.partial(_sc_body, l),
        out_type=jax.ShapeDtypeStruct((_NCORE, _NACC, D), jnp.float32),
        mesh=plsc.VectorSubcoreMesh(core_axis_name="c", subcore_axis_name="s",
                                    num_cores=_NCORE, num_subcores=_NSUB),
        scratch_types=[
            pltpu.VMEM((3, _CHUNK), jnp.int32),
            pltpu.VMEM((3, _CHUNK), jnp.int32),
            pltpu.VMEM((3, _CHUNK, D), jnp.float32),
            pltpu.VMEM_SHARED((_NACC, D), jnp.float32),
            pltpu.SemaphoreType.DMA((3,)),
            pltpu.SemaphoreType.DMA((3,)),
            pltpu.SemaphoreType.DMA((3,)),
        ],
    )(x, ee, src_pad, dst_pad)


_DL_BLK = 1000


def _dense_body(x_ref, a_ref, w1_ref, b1_ref, w2_ref, b2_ref,
                wih_ref, whh_ref, bih_ref, bhh_ref, o_ref):
    x = x_ref[...]
    t = x + a_ref[0] + a_ref[1]
    t = jnp.dot(t, w1_ref[...], preferred_element_type=jnp.float32) + b1_ref[...]
    t = _leaky(t * _BN)
    h = jnp.dot(t, w2_ref[...], preferred_element_type=jnp.float32) + b2_ref[...]
    h = jnp.where(h > 0, h, jnp.exp(h) - 1.0)
    gi = jnp.dot(h, wih_ref[...], preferred_element_type=jnp.float32) + bih_ref[...]
    gh = jnp.dot(x, whh_ref[...], preferred_element_type=jnp.float32) + bhh_ref[...]
    r = jax.nn.sigmoid(gi[:, :D] + gh[:, :D])
    z = jax.nn.sigmoid(gi[:, D:2 * D] + gh[:, D:2 * D])
    n = jnp.tanh(gi[:, 2 * D:] + r * gh[:, 2 * D:])
    o_ref[...] = _leaky((1.0 - z) * n + z * x)


def _dense_layer(x, aggr2, w1, b1, w2, b2, wih, whh, bih, bhh):
    full = lambda s: pl.BlockSpec(s, lambda i: tuple(0 for _ in s))
    return pl.pallas_call(
        _dense_body,
        grid=(N // _DL_BLK,),
        in_specs=[
            pl.BlockSpec((_DL_BLK, D), lambda i: (i, 0)),
            pl.BlockSpec((_NCORE, _DL_BLK, D), lambda i: (0, i, 0)),
            full((D, D)), full((1, D)), full((D, D)), full((1, D)),
            full((D, 3 * D)), full((D, 3 * D)), full((1, 3 * D)), full((1, 3 * D)),
        ],
        out_specs=pl.BlockSpec((_DL_BLK, D), lambda i: (i, 0)),
        out_shape=jax.ShapeDtypeStruct((N, D), jnp.float32),
    )(x, aggr2, w1, b1.reshape(1, D), w2, b2.reshape(1, D),
      wih, whh, bih.reshape(1, 3 * D), bhh.reshape(1, 3 * D))


def _readout_body(x_ref, b_ref, wl_ref, wr_ref, att_ref, bias_ref,
                  wih_ref, whh_ref, bih_ref, bhh_ref, lw_ref, lb_ref, o_ref):
    x = x_ref[...]
    oh = (b_ref[...] == lax.broadcasted_iota(jnp.int32, (N, G), 1)).astype(
        jnp.float32)
    dn = (((0,), (0,)), ((), ()))
    pool = lax.dot_general(oh, x, dn, preferred_element_type=jnp.float32)
    out = _leaky(pool)
    xl = jnp.dot(x, wl_ref[...], preferred_element_type=jnp.float32)
    att = att_ref[...]
    for _ in range(STEPS):
        xr = jnp.dot(out, wr_ref[...], preferred_element_type=jnp.float32)
        z = xl + jnp.dot(oh, xr, preferred_element_type=jnp.float32)
        z = jnp.where(z >= 0, z, 0.2 * z)
        e = jnp.sum(z * att, axis=1, keepdims=True)
        m = jnp.max(jnp.where(oh > 0, e, -jnp.inf), axis=0, keepdims=True)
        m = jnp.where(jnp.isfinite(m), m, 0.0)
        ex = jnp.exp(e - jnp.sum(oh * m, axis=1, keepdims=True))
        den = lax.dot_general(oh, ex, dn, preferred_element_type=jnp.float32)
        den_b = jnp.dot(oh, den, preferred_element_type=jnp.float32)
        alpha = ex / jnp.maximum(den_b, 1e-16)
        h = lax.dot_general(oh, alpha * xl, dn,
                            preferred_element_type=jnp.float32) + bias_ref[...]
        h = jnp.where(h > 0, h, jnp.exp(h) - 1.0)
        gi = jnp.dot(h, wih_ref[...], preferred_element_type=jnp.float32) \
            + bih_ref[...]
        gh = jnp.dot(out, whh_ref[...], preferred_element_type=jnp.float32) \
            + bhh_ref[...]
        r = jax.nn.sigmoid(gi[:, :D] + gh[:, :D])
        zz = jax.nn.sigmoid(gi[:, D:2 * D] + gh[:, D:2 * D])
        n = jnp.tanh(gi[:, 2 * D:] + r * gh[:, 2 * D:])
        out = _leaky((1.0 - zz) * n + zz * out)
    o_ref[...] = jnp.dot(out, lw_ref[...], preferred_element_type=jnp.float32) \
        + lb_ref[...]


def _readout(x, batch, mol_Wl, mol_Wr, mol_att, mol_bias,
             molgru_Wih, molgru_Whh, molgru_bih, molgru_bhh, lin_W, lin_b):
    full = lambda s: pl.BlockSpec(s, lambda: tuple(0 for _ in s))
    return pl.pallas_call(
        _readout_body,
        in_specs=[
            full((N, D)), full((N, 1)),
            full((D, D)), full((D, D)), full((1, D)), full((1, D)),
            full((D, 3 * D)), full((D, 3 * D)), full((1, 3 * D)),
            full((1, 3 * D)), full((D, D)), full((1, D)),
        ],
        out_specs=full((G, D)),
        out_shape=jax.ShapeDtypeStruct((G, D), jnp.float32),
    )(x, batch.reshape(N, 1), mol_Wl, mol_Wr, mol_att.reshape(1, D),
      mol_bias.reshape(1, D), molgru_Wih, molgru_Whh,
      molgru_bih.reshape(1, 3 * D), molgru_bhh.reshape(1, 3 * D),
      lin_W, lin_b.reshape(1, D))


def kernel(x, edge_index, edge_attr, batch, We, be, lin1_W, lin1_b, lin2_W,
           lin2_b, gru_Wih, gru_Whh, gru_bih, gru_bhh, mol_Wl, mol_Wr,
           mol_att, mol_bias, molgru_Wih, molgru_Whh, molgru_bih, molgru_bhh,
           lin_W, lin_b):
    src_pad = jnp.concatenate(
        [edge_index[0], jnp.zeros((_EP - E,), jnp.int32)])
    dst_pad = jnp.concatenate(
        [edge_index[1], jnp.full((_EP - E,), N, jnp.int32)])
    ea_pad = jnp.concatenate(
        [edge_attr, jnp.zeros((_EP - E, ED), jnp.float32)])
    e_emb3 = _edge_emb(ea_pad, We, be.reshape(L, 1, D))
    for l in range(L):
        aggr2 = _sc_edge(l, x, e_emb3, src_pad, dst_pad)
        x = _dense_layer(x, aggr2, lin1_W[l], lin1_b[l], lin2_W[l], lin2_b[l],
                         gru_Wih[l], gru_Whh[l], gru_bih[l], gru_bhh[l])
    return _readout(x, batch, mol_Wl, mol_Wr, mol_att, mol_bias,
                    molgru_Wih, molgru_Whh, molgru_bih, molgru_bhh,
                    lin_W, lin_b)


# speedup vs baseline: 2.3965x; 1.1139x over previous
import functools
import math

import jax
import jax.numpy as jnp
from jax import lax
from jax.experimental import pallas as pl
from jax.experimental.pallas import tpu as pltpu
from jax.experimental.pallas import tpu_sc as plsc

N = 10000
E = 320000
D = 128
ED = 16
L = 3
G = 128
STEPS = 2

_NCORE = 2
_NSUB = 16
_NW = _NCORE * _NSUB
_CHUNK = 128
_NCHUNK = 80
_EPW = _NCHUNK * _CHUNK
_EP = _EPW * _NW
_RPS = 632
_NACC = _RPS * _NSUB

_BN = 1.0 / math.sqrt(1.0 + 1e-5)


def _leaky(v, s=0.01):
    return jnp.where(v >= 0, v, s * v)


_EE_BLK = 8192


def _ee_body(ea_ref, w_ref, b_ref, out_ref):
    out_ref[0] = (
        jnp.dot(ea_ref[...], w_ref[0], preferred_element_type=jnp.float32)
        + b_ref[0]
    )


def _edge_emb(edge_attr_pad, We, be3):
    return pl.pallas_call(
        _ee_body,
        grid=(L, _EP // _EE_BLK),
        in_specs=[
            pl.BlockSpec((_EE_BLK, ED), lambda l, i: (i, 0)),
            pl.BlockSpec((1, ED, D), lambda l, i: (l, 0, 0)),
            pl.BlockSpec((1, 1, D), lambda l, i: (l, 0, 0)),
        ],
        out_specs=pl.BlockSpec((1, _EE_BLK, D), lambda l, i: (l, i, 0)),
        out_shape=jax.ShapeDtypeStruct((L, _EP, D), jnp.float32),
    )(edge_attr_pad, We, be3)


def _sc_body(l, x_hbm, ee_hbm, src_hbm, dst_hbm, out_hbm,
             srcv, dstv, buf, acc, sem_lin, sem_gat, sem_sc):
    c = lax.axis_index("c")
    s = lax.axis_index("s")
    woff = (c * _NSUB + s) * _EPW
    r0 = s * _RPS

    zero16 = jnp.zeros((16,), jnp.float32)

    def zrow(r, carry):
        for k in range(8):
            buf[0, r, pl.ds(k * 16, 16)] = zero16
        return carry

    lax.fori_loop(0, _CHUNK, zrow, 0)
    for j in range(4):
        pltpu.sync_copy(buf.at[0], acc.at[pl.ds(r0 + j * 128, 128)])
    pltpu.sync_copy(buf.at[0, pl.ds(0, _RPS - 512)],
                    acc.at[pl.ds(r0 + 512, _RPS - 512)])
    plsc.subcore_barrier()


    def lin_start(i, b):
        base = woff + i * _CHUNK
        pltpu.async_copy(src_hbm.at[pl.ds(base, _CHUNK)], srcv.at[b],
                         sem_lin.at[b])
        pltpu.async_copy(dst_hbm.at[pl.ds(base, _CHUNK)], dstv.at[b],
                         sem_lin.at[b])
        pltpu.async_copy(ee_hbm.at[l, pl.ds(base, _CHUNK)], buf.at[b],
                         sem_lin.at[b])

    def lin_wait(i, b):
        base = woff + i * _CHUNK
        pltpu.make_async_copy(src_hbm.at[pl.ds(base, _CHUNK)], srcv.at[b],
                              sem_lin.at[b]).wait()
        pltpu.make_async_copy(dst_hbm.at[pl.ds(base, _CHUNK)], dstv.at[b],
                              sem_lin.at[b]).wait()
        pltpu.make_async_copy(ee_hbm.at[l, pl.ds(base, _CHUNK)], buf.at[b],
                              sem_lin.at[b]).wait()

    def gat_start(b):
        pltpu.async_copy(x_hbm.at[srcv.at[b]], buf.at[b], sem_gat.at[b],
                         add=True)

    def gat_wait(b):
        pltpu.make_async_copy(x_hbm.at[srcv.at[b]], buf.at[b],
                              sem_gat.at[b]).wait()

    def relu(b):
        def relu_row(r, rcarry):
            for k in range(8):
                v = buf[b, r, pl.ds(k * 16, 16)]
                buf[b, r, pl.ds(k * 16, 16)] = jnp.maximum(v, 0.0)
            return rcarry

        lax.fori_loop(0, _CHUNK, relu_row, 0)

    def scat_start(b):
        pltpu.async_copy(buf.at[b], acc.at[dstv.at[b]], sem_sc.at[b],
                         add=True)

    def scat_wait(b):
        pltpu.make_async_copy(buf.at[b], acc.at[dstv.at[b]],
                              sem_sc.at[b]).wait()

    lin_start(0, 0)
    lin_start(1, 1)
    lin_wait(0, 0)
    gat_start(0)
    for i in (0, 1, 2):
        gat_wait(i)
        relu(i)
        scat_start(i)
        if i >= 1:
            scat_wait(i - 1)
        lin_start(i + 2, (i + 2) % 3)
        lin_wait(i + 1, (i + 1) % 3)
        gat_start((i + 1) % 3)

    def steady(k, carry):
        for j in range(3):
            i = 3 + 3 * k + j
            gat_wait(j)
            relu(j)
            scat_start(j)
            scat_wait((j + 2) % 3)
            lin_start(i + 2, (j + 2) % 3)
            lin_wait(i + 1, (j + 1) % 3)
            gat_start((j + 1) % 3)
        return carry

    lax.fori_loop(0, (_NCHUNK - 5) // 3, steady, 0)

    gat_wait(0)
    relu(0)
    scat_start(0)
    lin_wait(_NCHUNK - 1, 1)
    gat_start(1)
    gat_wait(1)
    relu(1)
    scat_start(1)
    scat_wait(2)
    scat_wait(0)
    scat_wait(1)
    plsc.subcore_barrier()

    for j in range(4):
        pltpu.sync_copy(acc.at[pl.ds(r0 + j * 128, 128)],
                        out_hbm.at[c, pl.ds(r0 + j * 128, 128)])
    pltpu.sync_copy(acc.at[pl.ds(r0 + 512, _RPS - 512)],
                    out_hbm.at[c, pl.ds(r0 + 512, _RPS - 512)])


def _sc_edge(l, x, ee, src_pad, dst_pad):
    return pl.kernel(
        functools.partial(_sc_body, l),
        out_type=jax.ShapeDtypeStruct((_NCORE, _NACC, D), jnp.float32),
        mesh=plsc.VectorSubcoreMesh(core_axis_name="c", subcore_axis_name="s",
                                    num_cores=_NCORE, num_subcores=_NSUB),
        scratch_types=[
            pltpu.VMEM((3, _CHUNK), jnp.int32),
            pltpu.VMEM((3, _CHUNK), jnp.int32),
            pltpu.VMEM((3, _CHUNK, D), jnp.float32),
            pltpu.VMEM_SHARED((_NACC, D), jnp.float32),
            pltpu.SemaphoreType.DMA((3,)),
            pltpu.SemaphoreType.DMA((3,)),
            pltpu.SemaphoreType.DMA((3,)),
        ],
    )(x, ee, src_pad, dst_pad)


_DL_BLK = 1000


def _dense_body(x_ref, a_ref, w1_ref, b1_ref, w2_ref, b2_ref,
                wih_ref, whh_ref, bih_ref, bhh_ref, o_ref):
    x = x_ref[...]
    t = x + a_ref[0] + a_ref[1]
    t = jnp.dot(t, w1_ref[...], preferred_element_type=jnp.float32) + b1_ref[...]
    t = _leaky(t * _BN)
    h = jnp.dot(t, w2_ref[...], preferred_element_type=jnp.float32) + b2_ref[...]
    h = jnp.where(h > 0, h, jnp.exp(h) - 1.0)
    gi = jnp.dot(h, wih_ref[...], preferred_element_type=jnp.float32) + bih_ref[...]
    gh = jnp.dot(x, whh_ref[...], preferred_element_type=jnp.float32) + bhh_ref[...]
    r = jax.nn.sigmoid(gi[:, :D] + gh[:, :D])
    z = jax.nn.sigmoid(gi[:, D:2 * D] + gh[:, D:2 * D])
    n = jnp.tanh(gi[:, 2 * D:] + r * gh[:, 2 * D:])
    o_ref[...] = _leaky((1.0 - z) * n + z * x)


def _dense_layer(x, aggr2, w1, b1, w2, b2, wih, whh, bih, bhh):
    full = lambda s: pl.BlockSpec(s, lambda i: tuple(0 for _ in s))
    return pl.pallas_call(
        _dense_body,
        grid=(N // _DL_BLK,),
        in_specs=[
            pl.BlockSpec((_DL_BLK, D), lambda i: (i, 0)),
            pl.BlockSpec((_NCORE, _DL_BLK, D), lambda i: (0, i, 0)),
            full((D, D)), full((1, D)), full((D, D)), full((1, D)),
            full((D, 3 * D)), full((D, 3 * D)), full((1, 3 * D)), full((1, 3 * D)),
        ],
        out_specs=pl.BlockSpec((_DL_BLK, D), lambda i: (i, 0)),
        out_shape=jax.ShapeDtypeStruct((N, D), jnp.float32),
    )(x, aggr2, w1, b1.reshape(1, D), w2, b2.reshape(1, D),
      wih, whh, bih.reshape(1, 3 * D), bhh.reshape(1, 3 * D))


def _readout_body(x_ref, b_ref, wl_ref, wr_ref, att_ref, bias_ref,
                  wih_ref, whh_ref, bih_ref, bhh_ref, lw_ref, lb_ref, o_ref):
    x = x_ref[...]
    oh = (b_ref[...] == lax.broadcasted_iota(jnp.int32, (N, G), 1)).astype(
        jnp.float32)
    dn = (((0,), (0,)), ((), ()))
    pool = lax.dot_general(oh, x, dn, preferred_element_type=jnp.float32)
    out = _leaky(pool)
    xl = jnp.dot(x, wl_ref[...], preferred_element_type=jnp.float32)
    att = att_ref[...]
    for _ in range(STEPS):
        xr = jnp.dot(out, wr_ref[...], preferred_element_type=jnp.float32)
        z = xl + jnp.dot(oh, xr, preferred_element_type=jnp.float32)
        z = jnp.where(z >= 0, z, 0.2 * z)
        e = jnp.sum(z * att, axis=1, keepdims=True)
        m = jnp.max(jnp.where(oh > 0, e, -jnp.inf), axis=0, keepdims=True)
        m = jnp.where(jnp.isfinite(m), m, 0.0)
        ex = jnp.exp(e - jnp.sum(oh * m, axis=1, keepdims=True))
        den = lax.dot_general(oh, ex, dn, preferred_element_type=jnp.float32)
        den_b = jnp.dot(oh, den, preferred_element_type=jnp.float32)
        alpha = ex / jnp.maximum(den_b, 1e-16)
        h = lax.dot_general(oh, alpha * xl, dn,
                            preferred_element_type=jnp.float32) + bias_ref[...]
        h = jnp.where(h > 0, h, jnp.exp(h) - 1.0)
        gi = jnp.dot(h, wih_ref[...], preferred_element_type=jnp.float32) \
            + bih_ref[...]
        gh = jnp.dot(out, whh_ref[...], preferred_element_type=jnp.float32) \
            + bhh_ref[...]
        r = jax.nn.sigmoid(gi[:, :D] + gh[:, :D])
        zz = jax.nn.sigmoid(gi[:, D:2 * D] + gh[:, D:2 * D])
        n = jnp.tanh(gi[:, 2 * D:] + r * gh[:, 2 * D:])
        out = _leaky((1.0 - zz) * n + zz * out)
    o_ref[...] = jnp.dot(out, lw_ref[...], preferred_element_type=jnp.float32) \
        + lb_ref[...]


def _readout(x, batch, mol_Wl, mol_Wr, mol_att, mol_bias,
             molgru_Wih, molgru_Whh, molgru_bih, molgru_bhh, lin_W, lin_b):
    full = lambda s: pl.BlockSpec(s, lambda: tuple(0 for _ in s))
    return pl.pallas_call(
        _readout_body,
        in_specs=[
            full((N, D)), full((N, 1)),
            full((D, D)), full((D, D)), full((1, D)), full((1, D)),
            full((D, 3 * D)), full((D, 3 * D)), full((1, 3 * D)),
            full((1, 3 * D)), full((D, D)), full((1, D)),
        ],
        out_specs=full((G, D)),
        out_shape=jax.ShapeDtypeStruct((G, D), jnp.float32),
    )(x, batch.reshape(N, 1), mol_Wl, mol_Wr, mol_att.reshape(1, D),
      mol_bias.reshape(1, D), molgru_Wih, molgru_Whh,
      molgru_bih.reshape(1, 3 * D), molgru_bhh.reshape(1, 3 * D),
      lin_W, lin_b.reshape(1, D))


def kernel(x, edge_index, edge_attr, batch, We, be, lin1_W, lin1_b, lin2_W,
           lin2_b, gru_Wih, gru_Whh, gru_bih, gru_bhh, mol_Wl, mol_Wr,
           mol_att, mol_bias, molgru_Wih, molgru_Whh, molgru_bih, molgru_bhh,
           lin_W, lin_b):
    src_pad = jnp.concatenate(
        [edge_index[0], jnp.zeros((_EP - E,), jnp.int32)])
    dst_pad = jnp.concatenate(
        [edge_index[1], jnp.full((_EP - E,), N, jnp.int32)])
    ea_pad = jnp.concatenate(
        [edge_attr, jnp.zeros((_EP - E, ED), jnp.float32)])
    e_emb3 = _edge_emb(ea_pad, We, be.reshape(L, 1, D))
    for l in range(L):
        aggr2 = _sc_edge(l, x, e_emb3, src_pad, dst_pad)
        x = _dense_layer(x, aggr2, lin1_W[l], lin1_b[l], lin2_W[l], lin2_b[l],
                         gru_Wih[l], gru_Whh[l], gru_bih[l], gru_bhh[l])
    return _readout(x, batch, mol_Wl, mol_Wr, mol_att, mol_bias,
                    molgru_Wih, molgru_Whh, molgru_bih, molgru_bhh,
                    lin_W, lin_b)
